# baseline (speedup 1.0000x reference)
import functools

import jax
import jax.numpy as jnp
from jax import lax
from jax.experimental import pallas as pl
from jax.experimental.pallas import tpu as pltpu
from jax.experimental.pallas import tpu_sc as plsc

N = 10000
E = 320000
D = 128
OUT = 128

NC = 2
NS = 16
L = 16
D2 = D + L
CH = 128
NP = 10240
RPT = NP // NS
WB = 128
KWB = RPT // WB
WBT = RPT - KWB * WB
NCHUNK = 161
GB = 7
NBLK = NCHUNK // GB
EPT = NCHUNK * CH
PE = EPT * NS
ERS = PE // CH
BR = 1000


_mesh = plsc.VectorSubcoreMesh(core_axis_name="c", subcore_axis_name="s")


@functools.partial(
    pl.kernel,
    out_type=jax.ShapeDtypeStruct((NC * NP, D2), jnp.float32),
    mesh=_mesh,
    scratch_types=[
        pltpu.VMEM_SHARED((NP, D2), jnp.float32),
        pltpu.VMEM((GB, CH), jnp.int32),
        pltpu.VMEM((GB, CH), jnp.int32),
        pltpu.VMEM((CH, D2), jnp.float32),
        pltpu.VMEM((CH, D2), jnp.float32),
        pltpu.SemaphoreType.DMA,
        pltpu.SemaphoreType.DMA,
    ],
    compiler_params=pltpu.CompilerParams(use_tc_tiling_on_sc=False),
)
def _sc_aggregate(feat_hbm, ei_hbm, sums_hbm, acc, gia, sia,
                  rows0, rows1, sem0, sem1):
    d = lax.axis_index("c")
    s = lax.axis_index("s")
    rb = s * RPT

    def zrow(r, carry):
        def zcol(c, carry2):
            rows0[r, pl.ds(c * L, L)] = jnp.zeros((L,), jnp.float32)
            return carry2
        lax.fori_loop(0, D2 // L, zcol, 0)
        return carry
    lax.fori_loop(0, WB, zrow, 0)
    for k in range(KWB):
        pltpu.sync_copy(rows0, acc.at[pl.ds(rb + k * WB, WB)])

    grow = d * ERS + s * NCHUNK
    scrow = (1 - d) * ERS + s * NCHUNK
    plsc.subcore_barrier()

    bufs = (rows0, rows1)
    sems = (sem0, sem1)

    def block(nb, carry):
        pltpu.sync_copy(ei_hbm.at[pl.ds(grow + nb * GB, GB)], gia)
        pltpu.sync_copy(ei_hbm.at[pl.ds(scrow + nb * GB, GB)], sia)
        pltpu.async_copy(feat_hbm.at[gia.at[0]], rows0, sem0)
        pltpu.async_copy(feat_hbm.at[gia.at[1]], rows1, sem1)
        for i in range(GB):
            b = i % 2
            pltpu.make_async_copy(feat_hbm.at[gia.at[i]], bufs[b],
                                  sems[b]).wait()
            pltpu.sync_copy(bufs[b], acc.at[sia.at[i]], add=True)
            if i + 2 < GB:
                pltpu.async_copy(feat_hbm.at[gia.at[i + 2]], bufs[b],
                                 sems[b])
        return carry
    lax.fori_loop(0, NBLK, block, 0)
    plsc.subcore_barrier()

    ob = d * NP + rb
    for k in range(KWB):
        pltpu.sync_copy(acc.at[pl.ds(rb + k * WB, WB)], rows0)
        pltpu.sync_copy(rows0, sums_hbm.at[pl.ds(ob + k * WB, WB)])


def _combine_body(feat_ref, sums_ref, w_ref, out_ref):
    w = w_ref[...]
    f = feat_ref[...][:, 0:D]
    s0 = sums_ref[0][:, 0:D]
    s1 = sums_ref[1][:, 0:D]
    d0 = jnp.maximum(sums_ref[0][:, D:D + 1], 1.0)
    d1 = jnp.maximum(sums_ref[1][:, D:D + 1], 1.0)
    acc = jnp.dot(f, w[0:D], preferred_element_type=jnp.float32)
    acc = acc + jnp.dot(s0 / d0, w[D:2 * D], preferred_element_type=jnp.float32)
    acc = acc + jnp.dot(s1 / d1, w[2 * D:3 * D], preferred_element_type=jnp.float32)
    out_ref[...] = acc


def kernel(feat, edge_index, W):
    featp = jnp.zeros((NP, D2), jnp.float32)
    featp = featp.at[:N, :D].set(feat)
    featp = featp.at[:, D].set(1.0)
    pad = jnp.full((2, PE - E), N, jnp.int32)
    eip = jnp.concatenate([edge_index, pad], axis=1)
    ei2 = eip.reshape(2 * ERS, CH)

    sums_f = _sc_aggregate(featp, ei2)
    sums = sums_f.reshape(NC, NP, D2)

    outp = pl.pallas_call(
        _combine_body,
        grid=(N // BR,),
        in_specs=[
            pl.BlockSpec((BR, D2), lambda j: (j, 0)),
            pl.BlockSpec((NC, BR, D2), lambda j: (0, j, 0)),
            pl.BlockSpec((3 * D, OUT), lambda j: (0, 0)),
        ],
        out_specs=pl.BlockSpec((BR, OUT), lambda j: (j, 0)),
        out_shape=jax.ShapeDtypeStruct((N, OUT), jnp.float32),
    )(featp, sums, W)
    return outp


# speedup vs baseline: 2.0724x; 2.0724x over previous
import functools

import jax
import jax.numpy as jnp
from jax import lax
from jax.experimental import pallas as pl
from jax.experimental.pallas import tpu as pltpu
from jax.experimental.pallas import tpu_sc as plsc

N = 10000
E = 320000
D = 128
OUT = 128

NC = 2
NS = 16
L = 16
D2 = D + L
CH = 112
NP = 10240
RPT = NP // NS
WB = 112
KWB = RPT // WB
WBT = RPT - KWB * WB
NCHUNK = 180
GB = 20
NBLK = NCHUNK // GB
EPT = NCHUNK * CH
PE = EPT * NS
ERS = PE // CH
BR = 1000


_mesh = plsc.VectorSubcoreMesh(core_axis_name="c", subcore_axis_name="s")


@functools.partial(
    pl.kernel,
    out_type=jax.ShapeDtypeStruct((NC * NP, D2), jnp.float32),
    mesh=_mesh,
    scratch_types=[
        pltpu.VMEM_SHARED((NP, D2), jnp.float32),
        pltpu.VMEM((GB, CH), jnp.int32),
        pltpu.VMEM((GB, CH), jnp.int32),
        pltpu.VMEM((CH, D2), jnp.float32),
        pltpu.VMEM((CH, D2), jnp.float32),
        pltpu.SemaphoreType.DMA,
        pltpu.SemaphoreType.DMA,
    ],
    compiler_params=pltpu.CompilerParams(use_tc_tiling_on_sc=False),
)
def _sc_aggregate(feat_hbm, ei_hbm, sums_hbm, acc, gia, sia,
                  rows0, rows1, sem0, sem1):
    d = lax.axis_index("c")
    s = lax.axis_index("s")
    rb = s * RPT

    def zrow(r, carry):
        def zcol(c, carry2):
            rows0[r, pl.ds(c * L, L)] = jnp.zeros((L,), jnp.float32)
            return carry2
        lax.fori_loop(0, D2 // L, zcol, 0)
        return carry
    lax.fori_loop(0, WB, zrow, 0)
    for k in range(KWB):
        pltpu.sync_copy(rows0, acc.at[pl.ds(rb + k * WB, WB)])
    pltpu.sync_copy(rows0.at[pl.ds(0, WBT)],
                    acc.at[pl.ds(rb + KWB * WB, WBT)])

    grow = d * ERS + s * NCHUNK
    scrow = (1 - d) * ERS + s * NCHUNK
    plsc.subcore_barrier()

    bufs = (rows0, rows1)
    sems = (sem0, sem1)

    def block(nb, carry):
        pltpu.sync_copy(ei_hbm.at[pl.ds(grow + nb * GB, GB)], gia)
        pltpu.sync_copy(ei_hbm.at[pl.ds(scrow + nb * GB, GB)], sia)
        pltpu.async_copy(feat_hbm.at[gia.at[0]], rows0, sem0)
        pltpu.async_copy(feat_hbm.at[gia.at[1]], rows1, sem1)
        for i in range(GB):
            b = i % 2
            pltpu.make_async_copy(feat_hbm.at[gia.at[i]], bufs[b],
                                  sems[b]).wait()
            pltpu.sync_copy(bufs[b], acc.at[sia.at[i]], add=True)
            if i + 2 < GB:
                pltpu.async_copy(feat_hbm.at[gia.at[i + 2]], bufs[b],
                                 sems[b])
        return carry
    lax.fori_loop(0, NBLK, block, 0)
    plsc.subcore_barrier()

    ob = d * NP + rb
    for k in range(KWB):
        pltpu.sync_copy(acc.at[pl.ds(rb + k * WB, WB)], rows0)
        pltpu.sync_copy(rows0, sums_hbm.at[pl.ds(ob + k * WB, WB)])
    pltpu.sync_copy(acc.at[pl.ds(rb + KWB * WB, WBT)],
                    rows0.at[pl.ds(0, WBT)])
    pltpu.sync_copy(rows0.at[pl.ds(0, WBT)],
                    sums_hbm.at[pl.ds(ob + KWB * WB, WBT)])


def _combine_body(feat_ref, sums_ref, w_ref, out_ref):
    w = w_ref[...]
    f = feat_ref[...][:, 0:D]
    s0 = sums_ref[0][:, 0:D]
    s1 = sums_ref[1][:, 0:D]
    d0 = jnp.maximum(sums_ref[0][:, D:D + 1], 1.0)
    d1 = jnp.maximum(sums_ref[1][:, D:D + 1], 1.0)
    acc = jnp.dot(f, w[0:D], preferred_element_type=jnp.float32)
    acc = acc + jnp.dot(s0 / d0, w[D:2 * D], preferred_element_type=jnp.float32)
    acc = acc + jnp.dot(s1 / d1, w[2 * D:3 * D], preferred_element_type=jnp.float32)
    out_ref[...] = acc


def kernel(feat, edge_index, W):
    featp = jnp.zeros((NP, D2), jnp.float32)
    featp = featp.at[:N, :D].set(feat)
    featp = featp.at[:, D].set(1.0)
    pad = jnp.full((2, PE - E), N, jnp.int32)
    eip = jnp.concatenate([edge_index, pad], axis=1)
    ei2 = eip.reshape(2 * ERS, CH)

    sums_f = _sc_aggregate(featp, ei2)
    sums = sums_f.reshape(NC, NP, D2)

    outp = pl.pallas_call(
        _combine_body,
        grid=(N // BR,),
        in_specs=[
            pl.BlockSpec((BR, D2), lambda j: (j, 0)),
            pl.BlockSpec((NC, BR, D2), lambda j: (0, j, 0)),
            pl.BlockSpec((3 * D, OUT), lambda j: (0, 0)),
        ],
        out_specs=pl.BlockSpec((BR, OUT), lambda j: (j, 0)),
        out_shape=jax.ShapeDtypeStruct((N, OUT), jnp.float32),
    )(featp, sums, W)
    return outp
